# Initial kernel scaffold; baseline (speedup 1.0000x reference)
import jax
import jax.numpy as jnp
from jax.experimental import pallas as pl


def kernel(x, W_enc, b_enc, W_dec, b_dec):
    raise NotImplementedError("write your pallas kernel here")


# speedup vs baseline: 9.8832x; 9.8832x over previous
import functools

import jax
import jax.numpy as jnp
from jax.experimental import pallas as pl

B = 256
T = 4
D_IN = 768
D_SAE = 16384
K_TOTAL = 128

BN_ENC = 512
ROWS_TK = 32
BK_DEC = 512


def _encode_kernel(x_ref, w_ref, b_ref, out_ref):
    acc = jnp.dot(x_ref[...], w_ref[...], preferred_element_type=jnp.float32)
    acc = acc + b_ref[...]
    out_ref[...] = jnp.where(acc > 0.0, acc, 0.0)


def _topk_kernel(pre_ref, z_ref):
    vals = pre_ref[...]
    bits = jax.lax.bitcast_convert_type(vals, jnp.int32)

    def body(_, carry):
        lo, hi = carry
        mid = lo + ((hi - lo) >> 1)
        cnt = jnp.sum((bits >= mid).astype(jnp.int32), axis=1, keepdims=True)
        take = cnt >= K_TOTAL
        lo = jnp.where(take, mid, lo)
        hi = jnp.where(take, hi, mid)
        return lo, hi

    rows = vals.shape[0]
    lo0 = jnp.zeros((rows, 1), jnp.int32)
    hi0 = jnp.full((rows, 1), jnp.int32(0x7FFFFFFF))
    lo, _ = jax.lax.fori_loop(0, 31, body, (lo0, hi0))
    keep = (bits >= lo) & (vals > 0.0)
    z_ref[...] = jnp.where(keep, vals, 0.0)


def _decode_kernel(z_ref, w_ref, b_ref, out_ref):
    k = pl.program_id(0)

    @pl.when(k == 0)
    def _init():
        out_ref[...] = jnp.broadcast_to(b_ref[...].reshape(1, T, D_IN), out_ref.shape)

    for t in range(T):
        acc = jnp.dot(z_ref[...], w_ref[t], preferred_element_type=jnp.float32)
        out_ref[:, t, :] += acc


@jax.jit
def kernel(x, W_enc, b_enc, W_dec, b_dec):
    x2 = x.reshape(B, T * D_IN)
    w_enc2 = W_enc.reshape(T * D_IN, D_SAE)
    b_enc2 = b_enc.reshape(1, D_SAE)

    pre = pl.pallas_call(
        _encode_kernel,
        grid=(D_SAE // BN_ENC,),
        in_specs=[
            pl.BlockSpec((B, T * D_IN), lambda j: (0, 0)),
            pl.BlockSpec((T * D_IN, BN_ENC), lambda j: (0, j)),
            pl.BlockSpec((1, BN_ENC), lambda j: (0, j)),
        ],
        out_specs=pl.BlockSpec((B, BN_ENC), lambda j: (0, j)),
        out_shape=jax.ShapeDtypeStruct((B, D_SAE), jnp.float32),
    )(x2, w_enc2, b_enc2)

    z = pl.pallas_call(
        _topk_kernel,
        grid=(B // ROWS_TK,),
        in_specs=[pl.BlockSpec((ROWS_TK, D_SAE), lambda i: (i, 0))],
        out_specs=pl.BlockSpec((ROWS_TK, D_SAE), lambda i: (i, 0)),
        out_shape=jax.ShapeDtypeStruct((B, D_SAE), jnp.float32),
    )(pre)

    x_hat = pl.pallas_call(
        _decode_kernel,
        grid=(D_SAE // BK_DEC,),
        in_specs=[
            pl.BlockSpec((B, BK_DEC), lambda k: (0, k)),
            pl.BlockSpec((T, BK_DEC, D_IN), lambda k: (0, k, 0)),
            pl.BlockSpec((T, D_IN), lambda k: (0, 0)),
        ],
        out_specs=pl.BlockSpec((B, T, D_IN), lambda k: (0, 0, 0)),
        out_shape=jax.ShapeDtypeStruct((B, T, D_IN), jnp.float32),
    )(z, W_dec, b_dec)

    return (x_hat, z)
